# Initial kernel scaffold; baseline (speedup 1.0000x reference)
import jax
import jax.numpy as jnp
from jax.experimental import pallas as pl


def kernel(X, Nbrs, Nbrs_Z, rc, rs, re):
    raise NotImplementedError("write your pallas kernel here")


# speedup vs baseline: 70.5902x; 70.5902x over previous
import functools
import math

import jax
import jax.numpy as jnp
from jax import lax
from jax.experimental import pallas as pl
from jax.experimental.pallas import tpu as pltpu
from jax.experimental.pallas import tpu_sc as plsc

_B, _N, _M, _D, _L = 8, 10000, 32, 3, 8
_NC, _NS = 2, 16
_QUARTERS = 4
_CH = 256
_R_BIG = 2560
_HALF_PI = math.pi / 2.0
_S3, _S5, _S7, _S9 = -1.0 / 6.0, 1.0 / 120.0, -1.0 / 5040.0, 1.0 / 362880.0


def _rsqrt_nr(x):
    i = lax.bitcast_convert_type(x, jnp.int32)
    i = jnp.int32(0x5F3759DF) - lax.shift_right_logical(i, 1)
    y = lax.bitcast_convert_type(i, jnp.float32)
    xh = x * 0.5
    for _ in range(3):
        y = y * (1.5 - xh * y * y)
    return y


def _sc_layer(X, Nbrs, Nbrs_Z, params):
    mesh = plsc.VectorSubcoreMesh(core_axis_name="c", subcore_axis_name="s")

    @functools.partial(
        pl.kernel,
        mesh=mesh,
        out_type=jax.ShapeDtypeStruct((_L, _B, _N), jnp.float32),
        compiler_params=pltpu.CompilerParams(
            needs_layout_passes=False, use_tc_tiling_on_sc=False),
        scratch_types=[
            pltpu.VMEM((_N, _D), jnp.float32),
            pltpu.VMEM((_CH, _M), jnp.int32),
            pltpu.VMEM((_CH, _M), jnp.int32),
            pltpu.VMEM((_L, _CH), jnp.float32),
            pltpu.VMEM((4, 16), jnp.float32),
        ],
    )
    def k(x_hbm, nbr_hbm, z_hbm, par_hbm, out_hbm, x_v, nbr_v, z_v, o_v, p_v):
        wid = lax.axis_index("c") * _NS + lax.axis_index("s")
        b = wid // _QUARTERS
        q = wid % _QUARTERS
        n0 = q * _R_BIG
        pltpu.sync_copy(par_hbm, p_v)
        pltpu.sync_copy(x_hbm.at[b], x_v)
        rc_row, rs_row, re_row, pirc_row = (p_v[i, :] for i in range(4))
        rc_s = [rc_row[l] for l in range(_L)]
        rs_s = [rs_row[l] for l in range(_L)]
        re_s = [re_row[l] for l in range(_L)]
        pirc_s = [pirc_row[l] for l in range(_L)]
        iota = lax.iota(jnp.int32, 16)
        c0 = jnp.zeros((16,), jnp.int32)
        c1 = jnp.full((16,), 1, jnp.int32)
        c2 = jnp.full((16,), 2, jnp.int32)
        zf = jnp.zeros((16,), jnp.float32)

        def process_chunk(base, cc):
            pltpu.sync_copy(nbr_hbm.at[b, pl.ds(base, cc)],
                            nbr_v.at[pl.ds(0, cc)])
            pltpu.sync_copy(z_hbm.at[b, pl.ds(base, cc)],
                            z_v.at[pl.ds(0, cc)])

            def group(g, carry):
                rows = g * 16 + iota
                arows = base + rows
                cx = plsc.load_gather(x_v, [arows, c0])
                cy = plsc.load_gather(x_v, [arows, c1])
                cz = plsc.load_gather(x_v, [arows, c2])

                def mstep(m, accs):
                    mv = jnp.full((16,), m, jnp.int32)
                    nid = plsc.load_gather(nbr_v, [rows, mv])
                    zz = plsc.load_gather(z_v, [rows, mv])
                    nx = plsc.load_gather(x_v, [nid, c0])
                    ny = plsc.load_gather(x_v, [nid, c1])
                    nz = plsc.load_gather(x_v, [nid, c2])
                    dx = nx - cx
                    dy = ny - cy
                    dz = nz - cz
                    r2 = dx * dx + dy * dy + dz * dz
                    r = r2 * _rsqrt_nr(r2)
                    zm = jnp.where(zz != 0, 1.0, 0.0).astype(jnp.float32)
                    out = []
                    for l in range(_L):
                        u = r * pirc_s[l] - _HALF_PI
                        z2 = u * u
                        p = _S9 * z2 + _S7
                        p = p * z2 + _S5
                        p = p * z2 + _S3
                        s = (p * z2) * u + u
                        fc = 0.5 - 0.5 * s
                        dd = r - rs_s[l]
                        e = jnp.exp(dd * dd * (-re_s[l]))
                        val = e * fc * zm
                        keep = r <= rc_s[l]
                        out.append(accs[l] + jnp.where(keep, val, zf))
                    return tuple(out)

                accs = lax.fori_loop(0, _M, mstep, (zf,) * _L)
                for l in range(_L):
                    o_v[l, pl.ds(g * 16, 16)] = accs[l]
                return carry

            lax.fori_loop(0, cc // 16, group, 0)
            for l in range(_L):
                pltpu.sync_copy(o_v.at[l, pl.ds(0, cc)],
                                out_hbm.at[l, b, pl.ds(base, cc)])

        def chunk(ci, carry):
            process_chunk(n0 + ci * _CH, _CH)
            return carry

        lax.fori_loop(0, 9, chunk, 0)

        @pl.when(q < 3)
        def _big_tail():
            process_chunk(n0 + 9 * _CH, _CH)

        @pl.when(q == 3)
        def _small_tail():
            process_chunk(n0 + 9 * _CH, 16)

    return k(X, Nbrs, Nbrs_Z, params)


def _batch_norm(layer):
    def body(x_ref, o_ref):
        x = x_ref[...]
        m = jnp.mean(x, axis=1, keepdims=True)
        d = x - m
        v = jnp.mean(d * d, axis=1, keepdims=True)
        o_ref[...] = d * lax.rsqrt(v + 0.001)

    return pl.pallas_call(
        body,
        out_shape=jax.ShapeDtypeStruct((_L, _B, _N), jnp.float32),
    )(layer)


def kernel(X, Nbrs, Nbrs_Z, rc, rs, re):
    rcf = rc.reshape(1, _L).astype(jnp.float32)
    rsf = rs.reshape(1, _L).astype(jnp.float32)
    ref = re.reshape(1, _L).astype(jnp.float32)
    pirc = math.pi / rcf
    params = jnp.pad(jnp.concatenate([rcf, rsf, ref, pirc], axis=0),
                     ((0, 0), (0, 16 - _L)))
    layer = _sc_layer(X, Nbrs.astype(jnp.int32), Nbrs_Z.astype(jnp.int32),
                      params)
    normed = _batch_norm(layer)
    return jnp.transpose(normed, (1, 2, 0))


# speedup vs baseline: 76.2609x; 1.0803x over previous
import functools
import math

import jax
import jax.numpy as jnp
from jax import lax
from jax.experimental import pallas as pl
from jax.experimental.pallas import tpu as pltpu
from jax.experimental.pallas import tpu_sc as plsc

_B, _N, _M, _D, _L = 8, 10000, 32, 3, 8
_NC, _NS = 2, 16
_CH = 256
_R_BIG = 2560
_PI = math.pi
_HALF_PI = math.pi / 2.0
_S1, _S3, _S5, _S7, _S9 = (9.9999997651e-01, -1.6666647593e-01,
                           8.3328992228e-03, -1.9800865307e-04,
                           2.5904300305e-06)
_FAR2 = 1e8


def _rsqrt_nr(x, iters=2):
    i = lax.bitcast_convert_type(x, jnp.int32)
    i = jnp.int32(0x5F3759DF) - lax.shift_right_logical(i, 1)
    y = lax.bitcast_convert_type(i, jnp.float32)
    xh = x * 0.5
    for _ in range(iters):
        y = y * (1.5 - xh * y * y)
    return y


def _sc_all(X, Nbrs, Nbrs_Z, params):
    mesh = plsc.VectorSubcoreMesh(core_axis_name="c", subcore_axis_name="s")

    @functools.partial(
        pl.kernel,
        mesh=mesh,
        out_type=jax.ShapeDtypeStruct((_B, _N, _L), jnp.float32),
        compiler_params=pltpu.CompilerParams(
            needs_layout_passes=False, use_tc_tiling_on_sc=False),
        scratch_types=[
            pltpu.VMEM((_N, _D), jnp.float32),
            pltpu.VMEM((_CH, _M), jnp.int32),
            pltpu.VMEM((_CH, _M), jnp.int32),
            pltpu.VMEM((_L, _CH), jnp.float32),
            pltpu.VMEM((4, 16), jnp.float32),
            pltpu.VMEM((_L, _B, 128), jnp.float32),
            pltpu.VMEM((128, _L), jnp.float32),
            pltpu.VMEM_SHARED((2, _L, _B, _R_BIG), jnp.float32),
        ],
    )
    def k(x_hbm, nbr_hbm, z_hbm, par_hbm, out_hbm,
          x_v, nbr_v, z_v, o_v, p_v, sh_v, t_v, shared):
        wid = lax.axis_index("c") * _NS + lax.axis_index("s")
        q = wid // _B
        b = wid % _B
        qq = q % 2
        n0 = q * _R_BIG
        pltpu.sync_copy(par_hbm, p_v)
        pltpu.sync_copy(x_hbm.at[b], x_v)
        pirc_row, c2_row, c1_row, c0_row = (p_v[i, :] for i in range(4))
        pirc_s = [pirc_row[l] for l in range(_L)]
        c2_s = [c2_row[l] for l in range(_L)]
        c1_s = [c1_row[l] for l in range(_L)]
        c0_s = [c0_row[l] for l in range(_L)]
        iota = lax.iota(jnp.int32, 16)
        c0 = jnp.zeros((16,), jnp.int32)
        c1 = jnp.full((16,), 1, jnp.int32)
        c2 = jnp.full((16,), 2, jnp.int32)
        zf = jnp.zeros((16,), jnp.float32)

        def layer_chunk(off, cc):
            base = n0 + off
            pltpu.sync_copy(nbr_hbm.at[b, pl.ds(base, cc)],
                            nbr_v.at[pl.ds(0, cc)])
            pltpu.sync_copy(z_hbm.at[b, pl.ds(base, cc)],
                            z_v.at[pl.ds(0, cc)])

            def group(g, carry):
                rows = g * 16 + iota
                arows = base + rows
                cx = plsc.load_gather(x_v, [arows, c0])
                cy = plsc.load_gather(x_v, [arows, c1])
                cz = plsc.load_gather(x_v, [arows, c2])

                def dist(mi):
                    mv = jnp.full((16,), 0, jnp.int32) + mi
                    nid = plsc.load_gather(nbr_v, [rows, mv])
                    zz = plsc.load_gather(z_v, [rows, mv])
                    nx = plsc.load_gather(x_v, [nid, c0])
                    ny = plsc.load_gather(x_v, [nid, c1])
                    nz = plsc.load_gather(x_v, [nid, c2])
                    dx = nx - cx
                    dy = ny - cy
                    dz = nz - cz
                    r2 = dx * dx + dy * dy + dz * dz
                    r2 = jnp.where(zz != 0, r2, _FAR2)
                    return r2 * _rsqrt_nr(r2), r2

                def mstep(mi, carry):
                    accs = carry[:_L]
                    r, r2 = carry[_L], carry[_L + 1]
                    rn, r2n = dist(jnp.minimum(mi + 1, _M - 1))
                    u = jnp.minimum(r * pirc_s[0], _PI) - _HALF_PI
                    z2 = u * u
                    p = _S9 * z2 + _S7
                    p = p * z2 + _S5
                    p = p * z2 + _S3
                    p = p * z2 + _S1
                    fc = 0.5 - 0.5 * (u * p)
                    out = []
                    for l in range(_L):
                        a = c2_s[l] * r2 + (c1_s[l] * r + c0_s[l])
                        out.append(accs[l] + jnp.exp(a) * fc)
                    return tuple(out) + (rn, r2n)

                r0, r20 = dist(0)
                accs = lax.fori_loop(0, _M, mstep, (zf,) * _L + (r0, r20))
                for l in range(_L):
                    o_v[l, pl.ds(g * 16, 16)] = accs[l]
                return carry

            lax.fori_loop(0, cc // 16, group, 0)
            pltpu.sync_copy(o_v.at[:, pl.ds(0, cc)],
                            shared.at[qq, :, b, pl.ds(off, cc)])

        def norm_chunk(off, cc):
            base = n0 + off
            pltpu.sync_copy(shared.at[qq, :, :, pl.ds(off, cc)],
                            sh_v.at[:, :, pl.ds(0, cc)])
            inv_b = 1.0 / _B
            bvec = jnp.full((16,), 0, jnp.int32) + b

            def nvec(v, carry):
                vs = v * 16
                for l in range(_L):
                    s = zf
                    s2 = zf
                    xb = []
                    for bb in range(_B):
                        xv = sh_v[l, bb, pl.ds(vs, 16)]
                        xb.append(xv)
                        s = s + xv
                        s2 = s2 + xv * xv
                    m = s * inv_b
                    var = s2 * inv_b - m * m
                    inv = _rsqrt_nr(jnp.maximum(var, 0.0) + 0.001, iters=3)
                    own = xb[0]
                    for bb in range(1, _B):
                        own = jnp.where(bvec == bb, xb[bb], own)
                    res = (own - m) * inv
                    plsc.store_scatter(t_v, [vs + iota, jnp.full((16,), l,
                                                                 jnp.int32)],
                                       res)
                return carry

            lax.fori_loop(0, cc // 16, nvec, 0)
            pltpu.sync_copy(t_v.at[pl.ds(0, cc)],
                            out_hbm.at[b, pl.ds(base, cc)])

        def chunk1(ci, carry):
            layer_chunk(ci * _CH, _CH)
            return carry

        lax.fori_loop(0, 9, chunk1, 0)

        @pl.when(q < 3)
        def _t1():
            layer_chunk(9 * _CH, _CH)

        @pl.when(q == 3)
        def _t2():
            layer_chunk(9 * _CH, 16)

        plsc.subcore_barrier()

        def chunk2(ci, carry):
            norm_chunk(ci * 128, 128)
            return carry

        lax.fori_loop(0, 18, chunk2, 0)

        @pl.when(q < 3)
        def _t3():
            norm_chunk(18 * 128, 128)
            norm_chunk(19 * 128, 128)

        @pl.when(q == 3)
        def _t4():
            norm_chunk(18 * 128, 16)

    return k(X, Nbrs, Nbrs_Z, params)


def kernel(X, Nbrs, Nbrs_Z, rc, rs, re):
    rcf = rc.reshape(1, _L).astype(jnp.float32)
    rsf = rs.reshape(1, _L).astype(jnp.float32)
    ref = re.reshape(1, _L).astype(jnp.float32)
    pirc = math.pi / rcf
    c2 = -ref
    c1 = 2.0 * ref * rsf
    c0 = -ref * rsf * rsf
    params = jnp.pad(jnp.concatenate([pirc, c2, c1, c0], axis=0),
                     ((0, 0), (0, 16 - _L)))
    return _sc_all(X, Nbrs.astype(jnp.int32), Nbrs_Z.astype(jnp.int32),
                   params)


# speedup vs baseline: 78.1763x; 1.0251x over previous
import functools
import math

import jax
import jax.numpy as jnp
from jax import lax
from jax.experimental import pallas as pl
from jax.experimental.pallas import tpu as pltpu
from jax.experimental.pallas import tpu_sc as plsc

_B, _N, _M, _D, _L = 8, 10000, 32, 3, 8
_NC, _NS = 2, 16
_QUARTERS = 4
_CH = 256
_R_BIG = 2560
_PI = math.pi
_HALF_PI = math.pi / 2.0
_S1, _S3, _S5, _S7, _S9 = (9.9999997651e-01, -1.6666647593e-01,
                           8.3328992228e-03, -1.9800865307e-04,
                           2.5904300305e-06)
_FAR2 = 1e8


def _rsqrt_nr(x):
    i = lax.bitcast_convert_type(x, jnp.int32)
    i = jnp.int32(0x5F3759DF) - lax.shift_right_logical(i, 1)
    y = lax.bitcast_convert_type(i, jnp.float32)
    xh = x * 0.5
    for _ in range(2):
        y = y * (1.5 - xh * y * y)
    return y


def _sc_layer(X, Nbrs, Nbrs_Z, params):
    mesh = plsc.VectorSubcoreMesh(core_axis_name="c", subcore_axis_name="s")

    @functools.partial(
        pl.kernel,
        mesh=mesh,
        out_type=jax.ShapeDtypeStruct((_L, _B, _N), jnp.float32),
        compiler_params=pltpu.CompilerParams(
            needs_layout_passes=False, use_tc_tiling_on_sc=False),
        scratch_types=[
            pltpu.VMEM((_N, _D), jnp.float32),
            pltpu.VMEM((_CH, _M), jnp.int32),
            pltpu.VMEM((_CH, _M), jnp.int32),
            pltpu.VMEM((_L, _CH), jnp.float32),
            pltpu.VMEM((4, 16), jnp.float32),
        ],
    )
    def k(x_hbm, nbr_hbm, z_hbm, par_hbm, out_hbm, x_v, nbr_v, z_v, o_v, p_v):
        wid = lax.axis_index("c") * _NS + lax.axis_index("s")
        b = wid // _QUARTERS
        q = wid % _QUARTERS
        n0 = q * _R_BIG
        pltpu.sync_copy(par_hbm, p_v)
        pltpu.sync_copy(x_hbm.at[b], x_v)
        pirc_row, c2_row, c1_row, c0_row = (p_v[i, :] for i in range(4))
        pirc_s = [pirc_row[l] for l in range(_L)]
        c2_s = [c2_row[l] for l in range(_L)]
        c1_s = [c1_row[l] for l in range(_L)]
        c0_s = [c0_row[l] for l in range(_L)]
        iota = lax.iota(jnp.int32, 16)
        c0 = jnp.zeros((16,), jnp.int32)
        c1 = jnp.full((16,), 1, jnp.int32)
        c2 = jnp.full((16,), 2, jnp.int32)
        zf = jnp.zeros((16,), jnp.float32)

        def process_chunk(base, cc):
            pltpu.sync_copy(nbr_hbm.at[b, pl.ds(base, cc)],
                            nbr_v.at[pl.ds(0, cc)])
            pltpu.sync_copy(z_hbm.at[b, pl.ds(base, cc)],
                            z_v.at[pl.ds(0, cc)])

            def group(g, carry):
                rows = g * 16 + iota
                arows = base + rows
                cx = plsc.load_gather(x_v, [arows, c0])
                cy = plsc.load_gather(x_v, [arows, c1])
                cz = plsc.load_gather(x_v, [arows, c2])

                def dist(mi):
                    mv = jnp.full((16,), 0, jnp.int32) + mi
                    nid = plsc.load_gather(nbr_v, [rows, mv])
                    zz = plsc.load_gather(z_v, [rows, mv])
                    nx = plsc.load_gather(x_v, [nid, c0])
                    ny = plsc.load_gather(x_v, [nid, c1])
                    nz = plsc.load_gather(x_v, [nid, c2])
                    dx = nx - cx
                    dy = ny - cy
                    dz = nz - cz
                    r2 = dx * dx + dy * dy + dz * dz
                    r2 = jnp.where(zz != 0, r2, _FAR2)
                    return r2 * _rsqrt_nr(r2), r2

                def mstep(mi, carry):
                    accs = carry[:_L]
                    r, r2 = carry[_L], carry[_L + 1]
                    rn, r2n = dist(jnp.minimum(mi + 1, _M - 1))
                    u = jnp.minimum(r * pirc_s[0], _PI) - _HALF_PI
                    z2 = u * u
                    p = _S9 * z2 + _S7
                    p = p * z2 + _S5
                    p = p * z2 + _S3
                    p = p * z2 + _S1
                    fc = 0.5 - 0.5 * (u * p)
                    out = []
                    for l in range(_L):
                        a = c2_s[l] * r2 + (c1_s[l] * r + c0_s[l])
                        out.append(accs[l] + jnp.exp(a) * fc)
                    return tuple(out) + (rn, r2n)

                r0, r20 = dist(0)
                accs = lax.fori_loop(0, _M, mstep, (zf,) * _L + (r0, r20))
                for l in range(_L):
                    o_v[l, pl.ds(g * 16, 16)] = accs[l]
                return carry

            lax.fori_loop(0, cc // 16, group, 0)
            pltpu.sync_copy(o_v.at[:, pl.ds(0, cc)],
                            out_hbm.at[:, b, pl.ds(base, cc)])

        def chunk(ci, carry):
            process_chunk(n0 + ci * _CH, _CH)
            return carry

        lax.fori_loop(0, 9, chunk, 0)

        @pl.when(q < 3)
        def _big_tail():
            process_chunk(n0 + 9 * _CH, _CH)

        @pl.when(q == 3)
        def _small_tail():
            process_chunk(n0 + 9 * _CH, 16)

    return k(X, Nbrs, Nbrs_Z, params)


def _batch_norm(layer):
    def body(x_ref, o_ref):
        x = x_ref[...]
        m = jnp.mean(x, axis=1, keepdims=True)
        d = x - m
        v = jnp.mean(d * d, axis=1, keepdims=True)
        o_ref[...] = jnp.transpose(d * lax.rsqrt(v + 0.001), (1, 2, 0))

    return pl.pallas_call(
        body,
        grid=(10,),
        in_specs=[pl.BlockSpec((_L, _B, 1024), lambda i: (0, 0, i))],
        out_specs=pl.BlockSpec((_B, 1024, _L), lambda i: (0, i, 0)),
        out_shape=jax.ShapeDtypeStruct((_B, _N, _L), jnp.float32),
    )(layer)


def kernel(X, Nbrs, Nbrs_Z, rc, rs, re):
    rcf = rc.reshape(1, _L).astype(jnp.float32)
    rsf = rs.reshape(1, _L).astype(jnp.float32)
    ref = re.reshape(1, _L).astype(jnp.float32)
    pirc = math.pi / rcf
    c2 = -ref
    c1 = 2.0 * ref * rsf
    c0 = -ref * rsf * rsf
    params = jnp.pad(jnp.concatenate([pirc, c2, c1, c0], axis=0),
                     ((0, 0), (0, 16 - _L)))
    layer = _sc_layer(X, Nbrs.astype(jnp.int32), Nbrs_Z.astype(jnp.int32),
                      params)
    return _batch_norm(layer)


# speedup vs baseline: 85.4235x; 1.0927x over previous
import functools
import math

import jax
import jax.numpy as jnp
from jax import lax
from jax.experimental import pallas as pl
from jax.experimental.pallas import tpu as pltpu
from jax.experimental.pallas import tpu_sc as plsc

_B, _N, _M, _D, _L = 8, 10000, 32, 3, 8
_NC, _NS = 2, 16
_QUARTERS = 4
_CH = 256
_R_BIG = 2560
_PI = math.pi
_HALF_PI = math.pi / 2.0
_S1, _S3, _S5, _S7, _S9 = (9.9999997651e-01, -1.6666647593e-01,
                           8.3328992228e-03, -1.9800865307e-04,
                           2.5904300305e-06)
_FAR2 = 1e8


def _rsqrt_nr(x):
    i = lax.bitcast_convert_type(x, jnp.int32)
    i = jnp.int32(0x5F3759DF) - lax.shift_right_logical(i, 1)
    y = lax.bitcast_convert_type(i, jnp.float32)
    xh = x * 0.5
    for _ in range(2):
        y = y * (1.5 - xh * y * y)
    return y


def _sc_layer(X, Nbrs, Nbrs_Z, params):
    mesh = plsc.VectorSubcoreMesh(core_axis_name="c", subcore_axis_name="s")

    @functools.partial(
        pl.kernel,
        mesh=mesh,
        out_type=jax.ShapeDtypeStruct((_L, _B, _N), jnp.float32),
        compiler_params=pltpu.CompilerParams(
            needs_layout_passes=False, use_tc_tiling_on_sc=False),
        scratch_types=[
            pltpu.VMEM((_N, _D), jnp.float32),
            pltpu.VMEM((_CH, _M), jnp.int32),
            pltpu.VMEM((_CH, _M), jnp.int32),
            pltpu.VMEM((_L, _CH), jnp.float32),
            pltpu.VMEM((4, 16), jnp.float32),
        ],
    )
    def k(x_hbm, nbr_hbm, z_hbm, par_hbm, out_hbm, x_v, nbr_v, z_v, o_v, p_v):
        wid = lax.axis_index("c") * _NS + lax.axis_index("s")
        b = wid // _QUARTERS
        q = wid % _QUARTERS
        n0 = q * _R_BIG
        pltpu.sync_copy(par_hbm, p_v)
        pltpu.sync_copy(x_hbm.at[b], x_v)
        pirc_row, c2_row, c1_row, c0_row = (p_v[i, :] for i in range(4))
        pirc_s = [pirc_row[l] for l in range(_L)]
        c2_s = [c2_row[l] for l in range(_L)]
        c1_s = [c1_row[l] for l in range(_L)]
        c0_s = [c0_row[l] for l in range(_L)]
        iota = lax.iota(jnp.int32, 16)
        c0 = jnp.zeros((16,), jnp.int32)
        c1 = jnp.full((16,), 1, jnp.int32)
        c2 = jnp.full((16,), 2, jnp.int32)
        zf = jnp.zeros((16,), jnp.float32)

        def process_chunk(base, cc):
            pltpu.sync_copy(nbr_hbm.at[b, pl.ds(base, cc)],
                            nbr_v.at[pl.ds(0, cc)])
            pltpu.sync_copy(z_hbm.at[b, pl.ds(base, cc)],
                            z_v.at[pl.ds(0, cc)])

            def group(g, carry):
                rows = g * 16 + iota
                arows = base + rows
                cx = plsc.load_gather(x_v, [arows, c0])
                cy = plsc.load_gather(x_v, [arows, c1])
                cz = plsc.load_gather(x_v, [arows, c2])

                def dist(mi):
                    mv = jnp.full((16,), 0, jnp.int32) + mi
                    nid = plsc.load_gather(nbr_v, [rows, mv])
                    zz = plsc.load_gather(z_v, [rows, mv])
                    nx = plsc.load_gather(x_v, [nid, c0])
                    ny = plsc.load_gather(x_v, [nid, c1])
                    nz = plsc.load_gather(x_v, [nid, c2])
                    dx = nx - cx
                    dy = ny - cy
                    dz = nz - cz
                    r2 = dx * dx + dy * dy + dz * dz
                    r2 = jnp.where(zz != 0, r2, _FAR2)
                    return r2 * _rsqrt_nr(r2), r2

                def mstep(mi, carry):
                    accs = carry[:_L]
                    r, r2 = carry[_L], carry[_L + 1]
                    rn, r2n = dist(jnp.minimum(mi + 1, _M - 1))
                    u = jnp.minimum(r * pirc_s[0], _PI) - _HALF_PI
                    z2 = u * u
                    p = _S9 * z2 + _S7
                    p = p * z2 + _S5
                    p = p * z2 + _S3
                    p = p * z2 + _S1
                    fc = 0.5 - 0.5 * (u * p)
                    out = []
                    for l in range(_L):
                        a = c2_s[l] * r2 + (c1_s[l] * r + c0_s[l])
                        out.append(accs[l] + jnp.exp(a) * fc)
                    return tuple(out) + (rn, r2n)

                r0, r20 = dist(0)
                accs = lax.fori_loop(0, _M, mstep, (zf,) * _L + (r0, r20))
                for l in range(_L):
                    o_v[l, pl.ds(g * 16, 16)] = accs[l]
                return carry

            lax.fori_loop(0, cc // 16, group, 0)
            pltpu.sync_copy(o_v.at[:, pl.ds(0, cc)],
                            out_hbm.at[:, b, pl.ds(base, cc)])

        def chunk(ci, carry):
            process_chunk(n0 + ci * _CH, _CH)
            return carry

        lax.fori_loop(0, 9, chunk, 0)

        @pl.when(q < 3)
        def _big_tail():
            process_chunk(n0 + 9 * _CH, _CH)

        @pl.when(q == 3)
        def _small_tail():
            process_chunk(n0 + 9 * _CH, 16)

    return k(X, Nbrs, Nbrs_Z, params)


def _batch_norm(layer):
    def body(x_ref, o_ref):
        x = x_ref[...]
        m = jnp.mean(x, axis=1, keepdims=True)
        d = x - m
        v = jnp.mean(d * d, axis=1, keepdims=True)
        o_ref[...] = d * lax.rsqrt(v + 0.001)

    return pl.pallas_call(
        body,
        out_shape=jax.ShapeDtypeStruct((_L, _B, _N), jnp.float32),
    )(layer)


def kernel(X, Nbrs, Nbrs_Z, rc, rs, re):
    rcf = rc.reshape(1, _L).astype(jnp.float32)
    rsf = rs.reshape(1, _L).astype(jnp.float32)
    ref = re.reshape(1, _L).astype(jnp.float32)
    pirc = math.pi / rcf
    c2 = -ref
    c1 = 2.0 * ref * rsf
    c0 = -ref * rsf * rsf
    params = jnp.pad(jnp.concatenate([pirc, c2, c1, c0], axis=0),
                     ((0, 0), (0, 16 - _L)))
    layer = _sc_layer(X, Nbrs.astype(jnp.int32), Nbrs_Z.astype(jnp.int32),
                      params)
    normed = _batch_norm(layer)
    return jnp.transpose(normed, (1, 2, 0))
